# baseline (speedup 1.0000x reference)
import functools

import jax
import jax.numpy as jnp
from jax import lax
from jax.experimental import pallas as pl
from jax.experimental.pallas import tpu as pltpu
from jax.experimental.pallas import tpu_sc as plsc

D = 768
L = 16
NK = D // L
N_TOK = 4096 * 32
K_ROWS = 64


def _rsqrt_vec(x):
    i = plsc.bitcast(x, jnp.int32)
    i = jnp.int32(0x5F3759DF) - lax.shift_right_logical(i, 1)
    y = plsc.bitcast(i, jnp.float32)
    for _ in range(3):
        y = y * (1.5 - 0.5 * x * y * y)
    return y


def _sc_body(n_workers, x_hbm, seg_hbm, tok_hbm, pos_hbm, segtab_hbm,
             gam_hbm, bet_hbm, out_hbm,
             idx_v, seg_v, buf, posp, dvec, gam, bet, stage, sem):
    tpw = N_TOK // n_workers
    n_chunks = tpw // K_ROWS
    wid = lax.axis_index("s") * 2 + lax.axis_index("c")
    base = wid * tpw

    pltpu.sync_copy(x_hbm.at[pl.ds(base, tpw)], idx_v)
    pltpu.sync_copy(seg_hbm.at[pl.ds(base, tpw)], seg_v)
    pltpu.sync_copy(gam_hbm, gam)
    pltpu.sync_copy(bet_hbm, bet)
    pltpu.sync_copy(segtab_hbm, stage)
    pltpu.sync_copy(pos_hbm, posp)

    for k in range(NK):
        sl = pl.ds(k * L, L)
        dvec[sl] = stage[pl.ds(D + k * L, L)] - stage[sl]

    def posfix(t, carry):
        for k in range(NK):
            sl = pl.ds(t * D + k * L, L)
            posp[sl] = posp[sl] + stage[pl.ds(k * L, L)]
        return carry

    lax.fori_loop(0, 32, posfix, 0)

    def chunk_body(c, carry):
        rowbase = c * K_ROWS
        copies = []
        for j in range(K_ROWS // L):
            iv = idx_v[pl.ds(rowbase + j * L, L)]
            copies.append(
                pltpu.async_copy(tok_hbm.at[iv], buf.at[pl.ds(j * L, L)], sem))
        for cp in copies:
            cp.wait()

        def row_body(r, rcarry):
            t = lax.rem(r, 32)
            sidx = jnp.full((L,), rowbase + r, jnp.int32)
            sv = plsc.load_gather(seg_v, [sidx])
            acc = jnp.zeros((L,), jnp.float32)
            acc2 = jnp.zeros((L,), jnp.float32)
            for k in range(NK):
                sl = pl.ds(k * L, L)
                tv = buf[r, sl] + posp[pl.ds(t * D + k * L, L)] + sv * dvec[sl]
                buf[r, sl] = tv
                acc = acc + tv
                acc2 = acc2 + tv * tv
            s1 = jnp.sum(acc)
            s2 = jnp.sum(acc2)
            mean = jnp.full((L,), s1, jnp.float32) * (1.0 / D)
            ex2 = jnp.full((L,), s2, jnp.float32) * (1.0 / D)
            inv = _rsqrt_vec(ex2 - mean * mean + 1e-5)
            for k in range(NK):
                sl = pl.ds(k * L, L)
                a = gam[sl] * inv
                b2 = bet[sl] - mean * a
                buf[r, sl] = buf[r, sl] * a + b2
            return rcarry

        lax.fori_loop(0, K_ROWS, row_body, 0)
        pltpu.sync_copy(buf, out_hbm.at[pl.ds(base + rowbase, K_ROWS)])
        return carry

    lax.fori_loop(0, n_chunks, chunk_body, 0)


@jax.jit
def kernel(x, seg, tok_table, pos_table, seg_table, gamma, beta):
    info = plsc.get_sparse_core_info()
    n_workers = info.num_cores * info.num_subcores
    tpw = N_TOK // n_workers
    mesh = plsc.VectorSubcoreMesh(core_axis_name="c", subcore_axis_name="s")
    run = pl.kernel(
        functools.partial(_sc_body, n_workers),
        mesh=mesh,
        compiler_params=pltpu.CompilerParams(needs_layout_passes=False),
        out_type=jax.ShapeDtypeStruct((N_TOK, D), jnp.float32),
        scratch_types=[
            pltpu.VMEM((tpw,), jnp.int32),
            pltpu.VMEM((tpw,), jnp.float32),
            pltpu.VMEM((K_ROWS, D), jnp.float32),
            pltpu.VMEM((32 * D,), jnp.float32),
            pltpu.VMEM((D,), jnp.float32),
            pltpu.VMEM((D,), jnp.float32),
            pltpu.VMEM((D,), jnp.float32),
            pltpu.VMEM((2 * D,), jnp.float32),
            pltpu.SemaphoreType.DMA,
        ],
    )
    out = run(x.reshape(-1), seg.astype(jnp.float32).reshape(-1),
              tok_table, pos_table.reshape(-1), seg_table.reshape(-1),
              gamma, beta)
    return out.reshape(x.shape[0], x.shape[1], D)
